# Initial kernel scaffold; baseline (speedup 1.0000x reference)
import functools

import jax
import jax.numpy as jnp
from jax import lax
from jax.experimental import pallas as pl
from jax.experimental.pallas import tpu as pltpu
from jax.experimental.pallas import tpu_sc as plsc

NC = 2
NS = 16
NW = NC * NS
CHUNK = 128

_mesh = plsc.VectorSubcoreMesh(core_axis_name="c", subcore_axis_name="s")


def _deg_body(n_pad, k, dst_hbm, ew_hbm, degp_hbm, dstv, ewv, deg_sh):
    cid = lax.axis_index("c")
    sid = lax.axis_index("s")
    wid = cid * NS + sid
    rows_per_sub = n_pad // NS

    for i in range(CHUNK // 16):
        ewv[0, pl.ds(i * 16, 16)] = jnp.zeros((16,), jnp.float32)

    @pl.loop(0, rows_per_sub // CHUNK)
    def _zcopy(i):
        pltpu.sync_copy(
            ewv.at[0],
            deg_sh.at[pl.ds(sid * rows_per_sub + i * CHUNK, CHUNK)],
        )

    plsc.subcore_barrier()

    pltpu.sync_copy(dst_hbm.at[wid], dstv)
    pltpu.sync_copy(ew_hbm.at[wid], ewv)

    @pl.loop(0, k)
    def _scatter(j):
        pltpu.sync_copy(ewv.at[j], deg_sh.at[dstv.at[j]], add=True)

    plsc.subcore_barrier()

    pltpu.sync_copy(
        deg_sh.at[pl.ds(sid * rows_per_sub, rows_per_sub)],
        degp_hbm.at[cid, pl.ds(sid * rows_per_sub, rows_per_sub)],
    )


def _make_deg_kernel(n_pad, k):
    return pl.kernel(
        functools.partial(_deg_body, n_pad, k),
        out_type=jax.ShapeDtypeStruct((NC, n_pad), jnp.float32),
        mesh=_mesh,
        scratch_types=[
            pltpu.VMEM((k, CHUNK), jnp.int32),
            pltpu.VMEM((k, CHUNK), jnp.float32),
            pltpu.VMEM_SHARED((n_pad,), jnp.float32),
        ],
    )


def _layer_body(n_pad, k, d, y_hbm, src_hbm, dst_hbm, ew_hbm, accp_hbm,
                srcv, dstv, ewv, rows, acc_sh, gsem):
    cid = lax.axis_index("c")
    sid = lax.axis_index("s")
    wid = cid * NS + sid
    rows_per_sub = n_pad // NS
    nq = d // 16

    @pl.loop(0, CHUNK)
    def _zr(r):
        for q in range(nq):
            rows[0, r, pl.ds(q * 16, 16)] = jnp.zeros((16,), jnp.float32)

    @pl.loop(0, rows_per_sub // CHUNK)
    def _zcopy(i):
        pltpu.sync_copy(
            rows.at[0],
            acc_sh.at[pl.ds(sid * rows_per_sub + i * CHUNK, CHUNK)],
        )

    plsc.subcore_barrier()

    pltpu.sync_copy(src_hbm.at[wid], srcv)
    pltpu.sync_copy(dst_hbm.at[wid], dstv)
    pltpu.sync_copy(ew_hbm.at[wid], ewv)

    def start_gather(j, b):
        pltpu.async_copy(y_hbm.at[srcv.at[j]], rows.at[b], gsem)

    def wait_gather(j, b):
        pltpu.make_async_copy(y_hbm.at[srcv.at[j]], rows.at[b], gsem).wait()

    start_gather(0, 0)
    start_gather(1, 1)

    @pl.loop(0, k // 2)
    def _outer(j0):
        for b in range(2):
            j = j0 * 2 + b
            wait_gather(j, b)

            @pl.loop(0, CHUNK)
            def _scale(e):
                s = ewv[j, e]
                for q in range(nq):
                    rows[b, e, pl.ds(q * 16, 16)] = rows[b, e, pl.ds(q * 16, 16)] * s

            pltpu.sync_copy(rows.at[b], acc_sh.at[dstv.at[j]], add=True)

            @pl.when(j + 2 < k)
            def _next():
                start_gather(j + 2, b)

    plsc.subcore_barrier()

    pltpu.sync_copy(
        acc_sh.at[pl.ds(sid * rows_per_sub, rows_per_sub)],
        accp_hbm.at[cid, pl.ds(sid * rows_per_sub, rows_per_sub)],
    )


def _make_layer_kernel(n_pad, k, d):
    return pl.kernel(
        functools.partial(_layer_body, n_pad, k, d),
        out_type=jax.ShapeDtypeStruct((NC, n_pad, d), jnp.float32),
        mesh=_mesh,
        scratch_types=[
            pltpu.VMEM((k, CHUNK), jnp.int32),
            pltpu.VMEM((k, CHUNK), jnp.int32),
            pltpu.VMEM((k, CHUNK), jnp.float32),
            pltpu.VMEM((2, CHUNK, d), jnp.float32),
            pltpu.VMEM_SHARED((n_pad, d), jnp.float32),
            pltpu.SemaphoreType.DMA,
        ],
    )


def _tc1_body(x_ref, w_ref, degp_ref, y_ref, dinv_ref):
    deg = degp_ref[:, 0:1] + degp_ref[:, 1:2] + 1.0
    dinv = lax.rsqrt(deg)
    xw = jnp.dot(x_ref[...], w_ref[...], preferred_element_type=jnp.float32)
    y_ref[...] = xw * dinv
    dinv_ref[...] = dinv


def _tc2_body(accp_ref, y_ref, dinv_ref, b_ref, w_ref, y2_ref):
    acc = accp_ref[0] + accp_ref[1] + y_ref[...]
    h = jnp.maximum(dinv_ref[...] * acc + b_ref[...], 0.0)
    y2_ref[...] = jnp.dot(h, w_ref[...], preferred_element_type=jnp.float32) * dinv_ref[...]


def _tc3_body(accp_ref, y_ref, dinv_ref, b_ref, out_ref):
    acc = accp_ref[0] + accp_ref[1] + y_ref[...]
    out_ref[...] = dinv_ref[...] * acc + b_ref[...]


def kernel(x, edge_index, edge_weight, W1, b1, W2, b2):
    n, d = x.shape
    e = edge_weight.shape[0]
    o = W2.shape[1]

    n_pad = ((n + NS * CHUNK - 1) // (NS * CHUNK)) * (NS * CHUNK)
    per_w = ((e + NW * CHUNK - 1) // (NW * CHUNK)) * CHUNK
    if (per_w // CHUNK) % 2:
        per_w += CHUNK
    e_pad = per_w * NW
    k = per_w // CHUNK

    src = jnp.concatenate([edge_index[0], jnp.zeros((e_pad - e,), jnp.int32)])
    dst = jnp.concatenate([edge_index[1], jnp.zeros((e_pad - e,), jnp.int32)])
    ew = jnp.concatenate([edge_weight, jnp.zeros((e_pad - e,), jnp.float32)])
    src_w = src.reshape(NW, k, CHUNK)
    dst_w = dst.reshape(NW, k, CHUNK)
    ew_w = ew.reshape(NW, k, CHUNK)
    x_pad = jnp.concatenate([x, jnp.zeros((n_pad - n, d), x.dtype)], axis=0)

    deg_kernel = _make_deg_kernel(n_pad, k)
    layer_kernel = _make_layer_kernel(n_pad, k, d)

    degp = deg_kernel(dst_w, ew_w)
    degp_t = degp.T

    y1, dinv = pl.pallas_call(
        _tc1_body,
        out_shape=[
            jax.ShapeDtypeStruct((n_pad, d), jnp.float32),
            jax.ShapeDtypeStruct((n_pad, 1), jnp.float32),
        ],
    )(x_pad, W1, degp_t)

    accp1 = layer_kernel(y1, src_w, dst_w, ew_w)

    y2 = pl.pallas_call(
        _tc2_body,
        out_shape=jax.ShapeDtypeStruct((n_pad, o), jnp.float32),
    )(accp1, y1, dinv, b1.reshape(1, -1), W2)

    accp2 = layer_kernel(y2, src_w, dst_w, ew_w)

    out = pl.pallas_call(
        _tc3_body,
        out_shape=jax.ShapeDtypeStruct((n_pad, o), jnp.float32),
    )(accp2, y2, dinv, b2.reshape(1, -1))

    return out[:n]


# speedup vs baseline: 10.6286x; 10.6286x over previous
import functools

import jax
import jax.numpy as jnp
from jax import lax
from jax.experimental import pallas as pl
from jax.experimental.pallas import tpu as pltpu
from jax.experimental.pallas import tpu_sc as plsc

NC = 2
NS = 16
NW = NC * NS
CHUNK = 128
GCH = 64
NBUF = 4
SB = 8

_mesh = plsc.VectorSubcoreMesh(core_axis_name="c", subcore_axis_name="s")


def _deg_body(n_pad, k, dst_hbm, ew_hbm, degp_hbm, dstv, ewv, deg_sh):
    cid = lax.axis_index("c")
    sid = lax.axis_index("s")
    wid = cid * NS + sid
    rows_per_sub = n_pad // NS

    for i in range(CHUNK // 16):
        ewv[0, pl.ds(i * 16, 16)] = jnp.zeros((16,), jnp.float32)

    @pl.loop(0, rows_per_sub // CHUNK)
    def _zcopy(i):
        pltpu.sync_copy(
            ewv.at[0],
            deg_sh.at[pl.ds(sid * rows_per_sub + i * CHUNK, CHUNK)],
        )

    plsc.subcore_barrier()

    pltpu.sync_copy(dst_hbm.at[wid], dstv)
    pltpu.sync_copy(ew_hbm.at[wid], ewv)

    @pl.loop(0, k)
    def _scatter(j):
        pltpu.sync_copy(ewv.at[j], deg_sh.at[dstv.at[j]], add=True)

    plsc.subcore_barrier()

    pltpu.sync_copy(
        deg_sh.at[pl.ds(sid * rows_per_sub, rows_per_sub)],
        degp_hbm.at[cid, pl.ds(sid * rows_per_sub, rows_per_sub)],
    )


def _make_deg_kernel(n_pad, k):
    return pl.kernel(
        functools.partial(_deg_body, n_pad, k),
        out_type=jax.ShapeDtypeStruct((NC, n_pad), jnp.float32),
        mesh=_mesh,
        scratch_types=[
            pltpu.VMEM((k, CHUNK), jnp.int32),
            pltpu.VMEM((k, CHUNK), jnp.float32),
            pltpu.VMEM_SHARED((n_pad,), jnp.float32),
        ],
    )


def _layer_body(n_pad, k, d, y_hbm, src_hbm, dst_hbm, ew_hbm, accp_hbm,
                srcv, dstv, ewv, rows, acc_sh, gsem, ssem, csem):
    cid = lax.axis_index("c")
    sid = lax.axis_index("s")
    wid = cid * NS + sid
    rows_per_sub = n_pad // NS
    nq = d // 16
    nb = k // SB

    @pl.loop(0, GCH)
    def _zr(r):
        for q in range(nq):
            rows[0, r, pl.ds(q * 16, 16)] = jnp.zeros((16,), jnp.float32)

    @pl.loop(0, rows_per_sub // GCH)
    def _zcopy(i):
        pltpu.sync_copy(
            rows.at[0],
            acc_sh.at[pl.ds(sid * rows_per_sub + i * GCH, GCH)],
        )

    plsc.subcore_barrier()

    def stage_start(bi, sl):
        pltpu.async_copy(src_hbm.at[wid, pl.ds(bi * SB, SB)], srcv.at[sl], ssem)
        pltpu.async_copy(dst_hbm.at[wid, pl.ds(bi * SB, SB)], dstv.at[sl], ssem)
        pltpu.async_copy(ew_hbm.at[wid, pl.ds(bi * SB, SB)], ewv.at[sl], ssem)

    def stage_wait(bi, sl):
        pltpu.make_async_copy(src_hbm.at[wid, pl.ds(bi * SB, SB)], srcv.at[sl], ssem).wait()
        pltpu.make_async_copy(dst_hbm.at[wid, pl.ds(bi * SB, SB)], dstv.at[sl], ssem).wait()
        pltpu.make_async_copy(ew_hbm.at[wid, pl.ds(bi * SB, SB)], ewv.at[sl], ssem).wait()

    def start_gather(j, b):
        sl = (j // SB) % 2
        pltpu.async_copy(y_hbm.at[srcv.at[sl, j % SB]], rows.at[b], gsem)

    def wait_gather(j, b):
        sl = (j // SB) % 2
        pltpu.make_async_copy(y_hbm.at[srcv.at[sl, j % SB]], rows.at[b], gsem).wait()

    def start_scatter(j, b):
        sl = (j // SB) % 2
        pltpu.async_copy(rows.at[b], acc_sh.at[dstv.at[sl, j % SB]], csem, add=True)

    def wait_scatter(j, b):
        sl = (j // SB) % 2
        pltpu.make_async_copy(rows.at[b], acc_sh.at[dstv.at[sl, j % SB]], csem).wait()

    stage_start(0, 0)
    stage_wait(0, 0)

    start_gather(0, 0)
    start_gather(1, 1)

    @pl.loop(0, k // NBUF)
    def _outer(j0):
        for b in range(NBUF):
            j = j0 * NBUF + b
            bi = j // SB
            jj = j % SB
            sl = bi % 2
            wait_gather(j, b)

            @pl.loop(0, GCH // 16)
            def _scale(g):
                ws = ewv[sl, jj, pl.ds(g * 16, 16)]
                for t in range(16):
                    s = ws[t]
                    for q in range(nq):
                        rows[b, g * 16 + t, pl.ds(q * 16, 16)] = (
                            rows[b, g * 16 + t, pl.ds(q * 16, 16)] * s)

            start_scatter(j, b)

            @pl.when(j >= 2)
            def _wsc():
                wait_scatter(j - 2, (j - 2) % NBUF)

            @pl.when(jnp.logical_and(jj == 1, bi + 1 < nb))
            def _rstage():
                stage_start(bi + 1, 1 - sl)

            @pl.when(jnp.logical_and(j + 2 < k, jj == SB - 2))
            def _wstage():
                stage_wait(bi + 1, 1 - sl)

            @pl.when(j + 2 < k)
            def _next():
                start_gather(j + 2, (j + 2) % NBUF)

    wait_scatter(k - 2, (k - 2) % NBUF)
    wait_scatter(k - 1, (k - 1) % NBUF)

    plsc.subcore_barrier()

    pltpu.sync_copy(
        acc_sh.at[pl.ds(sid * rows_per_sub, rows_per_sub)],
        accp_hbm.at[cid, pl.ds(sid * rows_per_sub, rows_per_sub)],
    )


def _make_layer_kernel(n_pad, k, d):
    return pl.kernel(
        functools.partial(_layer_body, n_pad, k, d),
        out_type=jax.ShapeDtypeStruct((NC, n_pad, d), jnp.float32),
        mesh=_mesh,
        scratch_types=[
            pltpu.VMEM((2, SB, GCH), jnp.int32),
            pltpu.VMEM((2, SB, GCH), jnp.int32),
            pltpu.VMEM((2, SB, GCH), jnp.float32),
            pltpu.VMEM((NBUF, GCH, d), jnp.float32),
            pltpu.VMEM_SHARED((n_pad, d), jnp.float32),
            pltpu.SemaphoreType.DMA,
            pltpu.SemaphoreType.DMA,
            pltpu.SemaphoreType.DMA,
        ],
    )


def _tc1_body(x_ref, w_ref, degp_ref, y_ref, dinv_ref):
    deg = degp_ref[:, 0:1] + degp_ref[:, 1:2] + 1.0
    dinv = lax.rsqrt(deg)
    xw = jnp.dot(x_ref[...], w_ref[...], preferred_element_type=jnp.float32)
    y_ref[...] = xw * dinv
    dinv_ref[...] = dinv


def _tc2_body(accp_ref, y_ref, dinv_ref, b_ref, w_ref, y2_ref):
    acc = accp_ref[0] + accp_ref[1] + y_ref[...]
    h = jnp.maximum(dinv_ref[...] * acc + b_ref[...], 0.0)
    y2_ref[...] = jnp.dot(h, w_ref[...], preferred_element_type=jnp.float32) * dinv_ref[...]


def _tc3_body(accp_ref, y_ref, dinv_ref, b_ref, out_ref):
    acc = accp_ref[0] + accp_ref[1] + y_ref[...]
    out_ref[...] = dinv_ref[...] * acc + b_ref[...]


def kernel(x, edge_index, edge_weight, W1, b1, W2, b2):
    n, d = x.shape
    e = edge_weight.shape[0]
    o = W2.shape[1]

    n_pad = ((n + NS * CHUNK - 1) // (NS * CHUNK)) * (NS * CHUNK)
    step = NW * GCH * 2 * SB
    assert step % (NW * CHUNK) == 0
    e_pad = ((e + step - 1) // step) * step
    per_w = e_pad // NW
    k = per_w // GCH
    kd = per_w // CHUNK

    src = jnp.concatenate([edge_index[0], jnp.zeros((e_pad - e,), jnp.int32)])
    dst = jnp.concatenate([edge_index[1], jnp.zeros((e_pad - e,), jnp.int32)])
    ew = jnp.concatenate([edge_weight, jnp.zeros((e_pad - e,), jnp.float32)])
    src_w = src.reshape(NW, k, GCH)
    dst_w = dst.reshape(NW, k, GCH)
    ew_w = ew.reshape(NW, k, GCH)
    dst_wd = dst.reshape(NW, kd, CHUNK)
    ew_wd = ew.reshape(NW, kd, CHUNK)
    x_pad = jnp.concatenate([x, jnp.zeros((n_pad - n, d), x.dtype)], axis=0)

    deg_kernel = _make_deg_kernel(n_pad, kd)
    layer_kernel = _make_layer_kernel(n_pad, k, d)

    degp = deg_kernel(dst_wd, ew_wd)
    degp_t = degp.T

    y1, dinv = pl.pallas_call(
        _tc1_body,
        out_shape=[
            jax.ShapeDtypeStruct((n_pad, d), jnp.float32),
            jax.ShapeDtypeStruct((n_pad, 1), jnp.float32),
        ],
    )(x_pad, W1, degp_t)

    accp1 = layer_kernel(y1, src_w, dst_w, ew_w)

    y2 = pl.pallas_call(
        _tc2_body,
        out_shape=jax.ShapeDtypeStruct((n_pad, o), jnp.float32),
    )(accp1, y1, dinv, b1.reshape(1, -1), W2)

    accp2 = layer_kernel(y2, src_w, dst_w, ew_w)

    out = pl.pallas_call(
        _tc3_body,
        out_shape=jax.ShapeDtypeStruct((n_pad, o), jnp.float32),
    )(accp2, y2, dinv, b2.reshape(1, -1))

    return out[:n]
